# baseline (device time: 60310 ns/iter reference)
import jax
import jax.numpy as jnp
from jax import lax
from jax.experimental import pallas as pl
from jax.experimental.pallas import tpu as pltpu

N_DEV = 32
E_LOCAL = 4
N_EXP = 128
N_TOK = 2048
D = 512
H = 1024
HH = H // 2
ROWS = N_TOK // N_DEV
NZ = 4
NQ = 8
SROWS = NZ * ROWS

CYC2Q = (0, 3, 4, 7, 6, 5, 2, 1)
Q2CYC = (0, 7, 6, 1, 2, 5, 4, 3)


def _lut(table, idx):
    r = 0
    for t, v in enumerate(table):
        r = r + v * (idx == t)
    return r


def kernel(x, router_W, route_idx, expert_W, shared_W):
    w_b = expert_W.astype(jnp.bfloat16)

    def body(x_ref, rw_ref, route_ref, w_ref, sw_ref, out_ref,
             xb_ref, gate4_ref, sh_ref,
             acc_r, acc_l, recv_r, recv_l, s_ref, acc2, recv2,
             s1r_send, s1r_recv, s1l_send, s1l_recv, s2_send, s2_recv):
        my = lax.axis_index("i")
        z = my // NQ
        q = my % NQ
        cm = _lut(Q2CYC, q)
        cyc_succ = z * NQ + _lut(CYC2Q, (cm + 1) % NQ)
        cyc_pred = z * NQ + _lut(CYC2Q, (cm - 1 + NQ) % NQ)
        zsucc = ((z + 1) % NZ) * NQ + q
        zpred = ((z - 1 + NZ) % NZ) * NQ + q

        xv = x_ref[...]
        scores = jnp.dot(xv, rw_ref[...], preferred_element_type=jnp.float32)
        smax = jnp.max(scores, axis=1, keepdims=True)
        e_ = jnp.exp(scores - smax)
        probs = e_ / jnp.sum(e_, axis=1, keepdims=True)
        route = route_ref[:, 0:1]
        oh128 = route == lax.broadcasted_iota(jnp.int32, (N_TOK, N_EXP), 1)
        p = jnp.sum(probs * oh128.astype(jnp.float32), axis=1, keepdims=True)
        gates = [
            (p * (route == (E_LOCAL * my + j)).astype(jnp.float32))
            for j in range(E_LOCAL)
        ]
        gate = jnp.concatenate(gates, axis=1)
        gate4_ref[...] = gate.astype(jnp.bfloat16).reshape(NZ, NQ * ROWS, E_LOCAL)
        xb_ref[...] = xv.astype(jnp.bfloat16).reshape(NZ, NQ * ROWS, D)

        def sc_partial_half(qq, lo):
            xz = xb_ref[:, pl.ds(qq * ROWS, ROWS), :]
            res = jnp.zeros((SROWS, HH), jnp.float32)
            for j in range(E_LOCAL):
                g = gate4_ref[:, pl.ds(qq * ROWS, ROWS), j:j + 1]
                xg = (xz * g).reshape(SROWS, D)
                res = res + jnp.dot(
                    xg, w_ref[j, :, lo:lo + HH],
                    preferred_element_type=jnp.float32,
                )
            return res

        q0r = _lut(CYC2Q, (cm - 1 + NQ) % NQ)
        q0l = _lut(CYC2Q, (cm + 1) % NQ)
        acc_r[0, :, :] = sc_partial_half(q0r, 0).astype(jnp.bfloat16)
        acc_l[0, :, :] = sc_partial_half(q0l, HH).astype(jnp.bfloat16)

        barrier_sem = pltpu.get_barrier_semaphore()
        for nbr in (cyc_pred, cyc_succ, zpred, zsucc):
            pl.semaphore_signal(
                barrier_sem, inc=1,
                device_id=(nbr,), device_id_type=pl.DeviceIdType.MESH,
            )
        pl.semaphore_wait(barrier_sem, 4)

        for s in range(NQ - 1):
            slot = s % 2
            rdma_r = pltpu.make_async_remote_copy(
                src_ref=acc_r.at[slot],
                dst_ref=recv_r.at[s],
                send_sem=s1r_send.at[s],
                recv_sem=s1r_recv.at[s],
                device_id=(cyc_succ,),
                device_id_type=pl.DeviceIdType.MESH,
            )
            rdma_l = pltpu.make_async_remote_copy(
                src_ref=acc_l.at[slot],
                dst_ref=recv_l.at[s],
                send_sem=s1l_send.at[s],
                recv_sem=s1l_recv.at[s],
                device_id=(cyc_pred,),
                device_id_type=pl.DeviceIdType.MESH,
            )
            rdma_r.start()
            rdma_l.start()
            qr = _lut(CYC2Q, (cm - s - 2 + 2 * NQ) % NQ)
            ql = _lut(CYC2Q, (cm + s + 2) % NQ)
            part_r = sc_partial_half(qr, 0)
            part_l = sc_partial_half(ql, HH)
            rdma_r.wait()
            rdma_l.wait()
            if s < NQ - 2:
                acc_r[1 - slot, :, :] = (
                    recv_r[s].astype(jnp.float32) + part_r
                ).astype(jnp.bfloat16)
                acc_l[1 - slot, :, :] = (
                    recv_l[s].astype(jnp.float32) + part_l
                ).astype(jnp.bfloat16)
            else:
                s_ref[:, 0:HH] = recv_r[s].astype(jnp.float32) + part_r
                s_ref[:, HH:H] = recv_l[s].astype(jnp.float32) + part_l

        p0 = (z - 1 + NZ) % NZ
        acc2[0, :, :] = s_ref[pl.ds(p0 * ROWS, ROWS), :].astype(jnp.bfloat16)
        for s in range(NZ - 1):
            slot = s % 2
            rdma2 = pltpu.make_async_remote_copy(
                src_ref=acc2.at[slot],
                dst_ref=recv2.at[s],
                send_sem=s2_send.at[s],
                recv_sem=s2_recv.at[s],
                device_id=(zsucc,),
                device_id_type=pl.DeviceIdType.MESH,
            )
            rdma2.start()
            if s == 0:
                sh_ref[...] = jnp.dot(
                    x_ref[pl.ds(my * ROWS, ROWS), :], sw_ref[...],
                    preferred_element_type=jnp.float32,
                )
            rdma2.wait()
            if s < NZ - 2:
                pz = (z - s - 2 + NZ) % NZ
                acc2[1 - slot, :, :] = (
                    recv2[s].astype(jnp.float32)
                    + s_ref[pl.ds(pz * ROWS, ROWS), :]
                ).astype(jnp.bfloat16)
            else:
                out_ref[...] = (
                    recv2[s].astype(jnp.float32)
                    + s_ref[pl.ds(z * ROWS, ROWS), :]
                    + sh_ref[...]
                )

    return pl.pallas_call(
        body,
        out_shape=jax.ShapeDtypeStruct((ROWS, H), jnp.float32),
        in_specs=[
            pl.BlockSpec(memory_space=pltpu.VMEM),
            pl.BlockSpec(memory_space=pltpu.VMEM),
            pl.BlockSpec(memory_space=pltpu.VMEM),
            pl.BlockSpec(memory_space=pltpu.VMEM),
            pl.BlockSpec(memory_space=pltpu.VMEM),
        ],
        out_specs=pl.BlockSpec(memory_space=pltpu.VMEM),
        scratch_shapes=[
            pltpu.VMEM((NZ, NQ * ROWS, D), jnp.bfloat16),
            pltpu.VMEM((NZ, NQ * ROWS, E_LOCAL), jnp.bfloat16),
            pltpu.VMEM((ROWS, H), jnp.float32),
            pltpu.VMEM((2, SROWS, HH), jnp.bfloat16),
            pltpu.VMEM((2, SROWS, HH), jnp.bfloat16),
            pltpu.VMEM((NQ - 1, SROWS, HH), jnp.bfloat16),
            pltpu.VMEM((NQ - 1, SROWS, HH), jnp.bfloat16),
            pltpu.VMEM((SROWS, H), jnp.float32),
            pltpu.VMEM((2, ROWS, H), jnp.bfloat16),
            pltpu.VMEM((NZ - 1, ROWS, H), jnp.bfloat16),
            pltpu.SemaphoreType.DMA((NQ - 1,)),
            pltpu.SemaphoreType.DMA((NQ - 1,)),
            pltpu.SemaphoreType.DMA((NQ - 1,)),
            pltpu.SemaphoreType.DMA((NQ - 1,)),
            pltpu.SemaphoreType.DMA((NZ - 1,)),
            pltpu.SemaphoreType.DMA((NZ - 1,)),
        ],
        compiler_params=pltpu.CompilerParams(collective_id=0),
    )(x, router_W, route_idx, w_b, shared_W)


# device time: 48536 ns/iter; 1.2426x vs baseline; 1.2426x over previous
import jax
import jax.numpy as jnp
from jax import lax
from jax.experimental import pallas as pl
from jax.experimental.pallas import tpu as pltpu

N_DEV = 32
E_LOCAL = 4
N_EXP = 128
N_TOK = 2048
D = 512
H = 1024
HH = H // 2
ROWS = N_TOK // N_DEV
NZ = 4
NQ = 8
SROWS = NZ * ROWS
SUBR = SROWS // 2

CYC2Q = (0, 3, 4, 7, 6, 5, 2, 1)
Q2CYC = (0, 7, 6, 1, 2, 5, 4, 3)


def _lut(table, idx):
    r = 0
    for t, v in enumerate(table):
        r = r + v * (idx == t)
    return r


def kernel(x, router_W, route_idx, expert_W, shared_W):
    w_b = expert_W.astype(jnp.bfloat16)

    def body(x_ref, rw_ref, route_ref, w_ref, sw_ref, out_ref,
             xb_ref, gate4_ref, sh_ref,
             acc_r, acc_l, recv_r, recv_l, s_ref, snd2, recv2,
             s1r_send, s1r_recv, s1l_send, s1l_recv, s2_send, s2_recv):
        my = lax.axis_index("i")
        z = my // NQ
        q = my % NQ
        cm = _lut(Q2CYC, q)
        cyc_succ = z * NQ + _lut(CYC2Q, (cm + 1) % NQ)
        cyc_pred = z * NQ + _lut(CYC2Q, (cm - 1 + NQ) % NQ)

        xv = x_ref[...]
        scores = jnp.dot(xv, rw_ref[...], preferred_element_type=jnp.float32)
        smax = jnp.max(scores, axis=1, keepdims=True)
        e_ = jnp.exp(scores - smax)
        probs = e_ / jnp.sum(e_, axis=1, keepdims=True)
        route = route_ref[:, 0:1]
        oh128 = route == lax.broadcasted_iota(jnp.int32, (N_TOK, N_EXP), 1)
        p = jnp.sum(probs * oh128.astype(jnp.float32), axis=1, keepdims=True)
        gates = [
            (p * (route == (E_LOCAL * my + j)).astype(jnp.float32))
            for j in range(E_LOCAL)
        ]
        gate = jnp.concatenate(gates, axis=1)
        gate4_ref[...] = gate.astype(jnp.bfloat16).reshape(NZ, NQ * ROWS, E_LOCAL)
        xb_ref[...] = xv.astype(jnp.bfloat16).reshape(NZ, NQ * ROWS, D)

        def sc_partial_half(qq, lo):
            xz = xb_ref[:, pl.ds(qq * ROWS, ROWS), :]
            res = jnp.zeros((SROWS, HH), jnp.float32)
            for j in range(E_LOCAL):
                g = gate4_ref[:, pl.ds(qq * ROWS, ROWS), j:j + 1]
                xg = (xz * g).reshape(SROWS, D)
                res = res + jnp.dot(
                    xg, w_ref[j, :, lo:lo + HH],
                    preferred_element_type=jnp.float32,
                )
            return res

        q0r = _lut(CYC2Q, (cm - 1 + NQ) % NQ)
        q0l = _lut(CYC2Q, (cm + 1) % NQ)
        acc_r[0, :, :, :] = (
            sc_partial_half(q0r, 0).astype(jnp.bfloat16).reshape(2, SUBR, HH)
        )
        acc_l[0, :, :, :] = (
            sc_partial_half(q0l, HH).astype(jnp.bfloat16).reshape(2, SUBR, HH)
        )

        barrier_sem = pltpu.get_barrier_semaphore()
        z_others = tuple(((z + dz) % NZ) * NQ + q for dz in (1, 2, 3))
        for nbr in (cyc_pred, cyc_succ) + z_others:
            pl.semaphore_signal(
                barrier_sem, inc=1,
                device_id=(nbr,), device_id_type=pl.DeviceIdType.MESH,
            )
        pl.semaphore_wait(barrier_sem, 5)

        def mk(ring, s, sub):
            acc, recv, ssend, srecv, dev = (
                (acc_r, recv_r, s1r_send, s1r_recv, cyc_succ)
                if ring == 0
                else (acc_l, recv_l, s1l_send, s1l_recv, cyc_pred)
            )
            return pltpu.make_async_remote_copy(
                src_ref=acc.at[s % 2, sub],
                dst_ref=recv.at[s, sub],
                send_sem=ssend.at[s, sub],
                recv_sem=srecv.at[s, sub],
                device_id=(dev,),
                device_id_type=pl.DeviceIdType.MESH,
            )

        mk(0, 0, 0).start()
        mk(1, 0, 0).start()
        mk(0, 0, 1).start()
        mk(1, 0, 1).start()
        for s in range(NQ - 1):
            slot = s % 2
            qr = _lut(CYC2Q, (cm - s - 2 + 2 * NQ) % NQ)
            ql = _lut(CYC2Q, (cm + s + 2) % NQ)
            part_r = sc_partial_half(qr, 0)
            part_l = sc_partial_half(ql, HH)
            for sub in range(2):
                lo, hi = sub * SUBR, (sub + 1) * SUBR
                rr = mk(0, s, sub)
                ll = mk(1, s, sub)
                rr.wait()
                ll.wait()
                if s < NQ - 2:
                    acc_r[1 - slot, sub, :, :] = (
                        recv_r[s, sub].astype(jnp.float32) + part_r[lo:hi]
                    ).astype(jnp.bfloat16)
                    acc_l[1 - slot, sub, :, :] = (
                        recv_l[s, sub].astype(jnp.float32) + part_l[lo:hi]
                    ).astype(jnp.bfloat16)
                    mk(0, s + 1, sub).start()
                    mk(1, s + 1, sub).start()
                else:
                    s_ref[lo:hi, 0:HH] = (
                        recv_r[s, sub].astype(jnp.float32) + part_r[lo:hi]
                    )
                    s_ref[lo:hi, HH:H] = (
                        recv_l[s, sub].astype(jnp.float32) + part_l[lo:hi]
                    )

        p2 = []
        for dz in (1, 2, 3):
            zt = (z + dz) % NZ
            snd2[dz - 1, :, :] = (
                s_ref[pl.ds(zt * ROWS, ROWS), :].astype(jnp.bfloat16)
            )
            rdma2 = pltpu.make_async_remote_copy(
                src_ref=snd2.at[dz - 1],
                dst_ref=recv2.at[dz - 1],
                send_sem=s2_send.at[dz - 1],
                recv_sem=s2_recv.at[dz - 1],
                device_id=(zt * NQ + q,),
                device_id_type=pl.DeviceIdType.MESH,
            )
            rdma2.start()
            p2.append(rdma2)
        sh_ref[...] = jnp.dot(
            x_ref[pl.ds(my * ROWS, ROWS), :], sw_ref[...],
            preferred_element_type=jnp.float32,
        )
        for rdma2 in p2:
            rdma2.wait()
        out_ref[...] = (
            s_ref[pl.ds(z * ROWS, ROWS), :]
            + sh_ref[...]
            + recv2[0].astype(jnp.float32)
            + recv2[1].astype(jnp.float32)
            + recv2[2].astype(jnp.float32)
        )

    return pl.pallas_call(
        body,
        out_shape=jax.ShapeDtypeStruct((ROWS, H), jnp.float32),
        in_specs=[
            pl.BlockSpec(memory_space=pltpu.VMEM),
            pl.BlockSpec(memory_space=pltpu.VMEM),
            pl.BlockSpec(memory_space=pltpu.VMEM),
            pl.BlockSpec(memory_space=pltpu.VMEM),
            pl.BlockSpec(memory_space=pltpu.VMEM),
        ],
        out_specs=pl.BlockSpec(memory_space=pltpu.VMEM),
        scratch_shapes=[
            pltpu.VMEM((NZ, NQ * ROWS, D), jnp.bfloat16),
            pltpu.VMEM((NZ, NQ * ROWS, E_LOCAL), jnp.bfloat16),
            pltpu.VMEM((ROWS, H), jnp.float32),
            pltpu.VMEM((2, 2, SUBR, HH), jnp.bfloat16),
            pltpu.VMEM((2, 2, SUBR, HH), jnp.bfloat16),
            pltpu.VMEM((NQ - 1, 2, SUBR, HH), jnp.bfloat16),
            pltpu.VMEM((NQ - 1, 2, SUBR, HH), jnp.bfloat16),
            pltpu.VMEM((SROWS, H), jnp.float32),
            pltpu.VMEM((NZ - 1, ROWS, H), jnp.bfloat16),
            pltpu.VMEM((NZ - 1, ROWS, H), jnp.bfloat16),
            pltpu.SemaphoreType.DMA((NQ - 1, 2)),
            pltpu.SemaphoreType.DMA((NQ - 1, 2)),
            pltpu.SemaphoreType.DMA((NQ - 1, 2)),
            pltpu.SemaphoreType.DMA((NQ - 1, 2)),
            pltpu.SemaphoreType.DMA((NZ - 1,)),
            pltpu.SemaphoreType.DMA((NZ - 1,)),
        ],
        compiler_params=pltpu.CompilerParams(collective_id=0),
    )(x, router_W, route_idx, w_b, shared_W)


# device time: 48184 ns/iter; 1.2517x vs baseline; 1.0073x over previous
import jax
import jax.numpy as jnp
from jax import lax
from jax.experimental import pallas as pl
from jax.experimental.pallas import tpu as pltpu

N_DEV = 32
E_LOCAL = 4
N_EXP = 128
N_TOK = 2048
D = 512
H = 1024
HH = H // 2
ROWS = N_TOK // N_DEV
NZ = 4
NQ = 8
SROWS = NZ * ROWS
NSUB = 4
SUBR = SROWS // NSUB

CYC2Q = (0, 3, 4, 7, 6, 5, 2, 1)
Q2CYC = (0, 7, 6, 1, 2, 5, 4, 3)


def _lut(table, idx):
    r = 0
    for t, v in enumerate(table):
        r = r + v * (idx == t)
    return r


def kernel(x, router_W, route_idx, expert_W, shared_W):
    w_b = expert_W.astype(jnp.bfloat16)

    def body(x_ref, rw_ref, route_ref, w_ref, sw_ref, out_ref,
             xb_ref, gate4_ref, sh_ref,
             acc_r, acc_l, recv_r, recv_l, s_ref, snd2, recv2,
             s1r_send, s1r_recv, s1l_send, s1l_recv, s2_send, s2_recv):
        my = lax.axis_index("i")
        z = my // NQ
        q = my % NQ
        cm = _lut(Q2CYC, q)
        cyc_succ = z * NQ + _lut(CYC2Q, (cm + 1) % NQ)
        cyc_pred = z * NQ + _lut(CYC2Q, (cm - 1 + NQ) % NQ)

        xv = x_ref[...]
        scores = jnp.dot(xv, rw_ref[...], preferred_element_type=jnp.float32)
        smax = jnp.max(scores, axis=1, keepdims=True)
        e_ = jnp.exp(scores - smax)
        probs = e_ / jnp.sum(e_, axis=1, keepdims=True)
        route = route_ref[:, 0:1]
        oh128 = route == lax.broadcasted_iota(jnp.int32, (N_TOK, N_EXP), 1)
        p = jnp.sum(probs * oh128.astype(jnp.float32), axis=1, keepdims=True)
        gates = [
            (p * (route == (E_LOCAL * my + j)).astype(jnp.float32))
            for j in range(E_LOCAL)
        ]
        gate = jnp.concatenate(gates, axis=1)
        gate4_ref[...] = gate.astype(jnp.bfloat16).reshape(NZ, NQ * ROWS, E_LOCAL)
        xb_ref[...] = xv.astype(jnp.bfloat16).reshape(NZ, NQ * ROWS, D)

        def sc_partial_half(qq, lo):
            xz = xb_ref[:, pl.ds(qq * ROWS, ROWS), :]
            res = jnp.zeros((SROWS, HH), jnp.float32)
            for j in range(E_LOCAL):
                g = gate4_ref[:, pl.ds(qq * ROWS, ROWS), j:j + 1]
                xg = (xz * g).reshape(SROWS, D)
                res = res + jnp.dot(
                    xg, w_ref[j, :, lo:lo + HH],
                    preferred_element_type=jnp.float32,
                )
            return res

        q0r = _lut(CYC2Q, (cm - 1 + NQ) % NQ)
        q0l = _lut(CYC2Q, (cm + 1) % NQ)
        acc_r[0, :, :, :] = (
            sc_partial_half(q0r, 0).astype(jnp.bfloat16).reshape(NSUB, SUBR, HH)
        )
        acc_l[0, :, :, :] = (
            sc_partial_half(q0l, HH).astype(jnp.bfloat16).reshape(NSUB, SUBR, HH)
        )

        barrier_sem = pltpu.get_barrier_semaphore()
        z_others = tuple(((z + dz) % NZ) * NQ + q for dz in (1, 2, 3))
        for nbr in (cyc_pred, cyc_succ) + z_others:
            pl.semaphore_signal(
                barrier_sem, inc=1,
                device_id=(nbr,), device_id_type=pl.DeviceIdType.MESH,
            )
        pl.semaphore_wait(barrier_sem, 5)

        def mk(ring, s, sub):
            acc, recv, ssend, srecv, dev = (
                (acc_r, recv_r, s1r_send, s1r_recv, cyc_succ)
                if ring == 0
                else (acc_l, recv_l, s1l_send, s1l_recv, cyc_pred)
            )
            return pltpu.make_async_remote_copy(
                src_ref=acc.at[s % 2, sub],
                dst_ref=recv.at[s, sub],
                send_sem=ssend.at[s, sub],
                recv_sem=srecv.at[s, sub],
                device_id=(dev,),
                device_id_type=pl.DeviceIdType.MESH,
            )

        for sub in range(NSUB):
            mk(0, 0, sub).start()
            mk(1, 0, sub).start()
        for s in range(NQ - 1):
            slot = s % 2
            qr = _lut(CYC2Q, (cm - s - 2 + 2 * NQ) % NQ)
            ql = _lut(CYC2Q, (cm + s + 2) % NQ)
            part_r = sc_partial_half(qr, 0)
            part_l = sc_partial_half(ql, HH)
            for sub in range(NSUB):
                lo, hi = sub * SUBR, (sub + 1) * SUBR
                rr = mk(0, s, sub)
                ll = mk(1, s, sub)
                rr.wait()
                ll.wait()
                if s < NQ - 2:
                    acc_r[1 - slot, sub, :, :] = (
                        recv_r[s, sub].astype(jnp.float32) + part_r[lo:hi]
                    ).astype(jnp.bfloat16)
                    acc_l[1 - slot, sub, :, :] = (
                        recv_l[s, sub].astype(jnp.float32) + part_l[lo:hi]
                    ).astype(jnp.bfloat16)
                    mk(0, s + 1, sub).start()
                    mk(1, s + 1, sub).start()
                else:
                    s_ref[lo:hi, 0:HH] = (
                        recv_r[s, sub].astype(jnp.float32) + part_r[lo:hi]
                    )
                    s_ref[lo:hi, HH:H] = (
                        recv_l[s, sub].astype(jnp.float32) + part_l[lo:hi]
                    )

        p2 = []
        for dz in (1, 2, 3):
            zt = (z + dz) % NZ
            snd2[dz - 1, :, :] = (
                s_ref[pl.ds(zt * ROWS, ROWS), :].astype(jnp.bfloat16)
            )
            rdma2 = pltpu.make_async_remote_copy(
                src_ref=snd2.at[dz - 1],
                dst_ref=recv2.at[dz - 1],
                send_sem=s2_send.at[dz - 1],
                recv_sem=s2_recv.at[dz - 1],
                device_id=(zt * NQ + q,),
                device_id_type=pl.DeviceIdType.MESH,
            )
            rdma2.start()
            p2.append(rdma2)
        sh_ref[...] = jnp.dot(
            x_ref[pl.ds(my * ROWS, ROWS), :], sw_ref[...],
            preferred_element_type=jnp.float32,
        )
        for rdma2 in p2:
            rdma2.wait()
        out_ref[...] = (
            s_ref[pl.ds(z * ROWS, ROWS), :]
            + sh_ref[...]
            + recv2[0].astype(jnp.float32)
            + recv2[1].astype(jnp.float32)
            + recv2[2].astype(jnp.float32)
        )

    return pl.pallas_call(
        body,
        out_shape=jax.ShapeDtypeStruct((ROWS, H), jnp.float32),
        in_specs=[
            pl.BlockSpec(memory_space=pltpu.VMEM),
            pl.BlockSpec(memory_space=pltpu.VMEM),
            pl.BlockSpec(memory_space=pltpu.VMEM),
            pl.BlockSpec(memory_space=pltpu.VMEM),
            pl.BlockSpec(memory_space=pltpu.VMEM),
        ],
        out_specs=pl.BlockSpec(memory_space=pltpu.VMEM),
        scratch_shapes=[
            pltpu.VMEM((NZ, NQ * ROWS, D), jnp.bfloat16),
            pltpu.VMEM((NZ, NQ * ROWS, E_LOCAL), jnp.bfloat16),
            pltpu.VMEM((ROWS, H), jnp.float32),
            pltpu.VMEM((2, NSUB, SUBR, HH), jnp.bfloat16),
            pltpu.VMEM((2, NSUB, SUBR, HH), jnp.bfloat16),
            pltpu.VMEM((NQ - 1, NSUB, SUBR, HH), jnp.bfloat16),
            pltpu.VMEM((NQ - 1, NSUB, SUBR, HH), jnp.bfloat16),
            pltpu.VMEM((SROWS, H), jnp.float32),
            pltpu.VMEM((NZ - 1, ROWS, H), jnp.bfloat16),
            pltpu.VMEM((NZ - 1, ROWS, H), jnp.bfloat16),
            pltpu.SemaphoreType.DMA((NQ - 1, NSUB)),
            pltpu.SemaphoreType.DMA((NQ - 1, NSUB)),
            pltpu.SemaphoreType.DMA((NQ - 1, NSUB)),
            pltpu.SemaphoreType.DMA((NQ - 1, NSUB)),
            pltpu.SemaphoreType.DMA((NZ - 1,)),
            pltpu.SemaphoreType.DMA((NZ - 1,)),
        ],
        compiler_params=pltpu.CompilerParams(collective_id=0),
    )(x, router_W, route_idx, w_b, shared_W)


# device time: 46036 ns/iter; 1.3101x vs baseline; 1.0467x over previous
import jax
import jax.numpy as jnp
from jax import lax
from jax.experimental import pallas as pl
from jax.experimental.pallas import tpu as pltpu

N_DEV = 32
E_LOCAL = 4
N_EXP = 128
N_TOK = 2048
D = 512
H = 1024
HH = H // 2
ROWS = N_TOK // N_DEV
NZ = 4
NQ = 8
SROWS = NZ * ROWS
NSUB = 4
SUBR = SROWS // NSUB

CYC2Q = (0, 3, 4, 7, 6, 5, 2, 1)
Q2CYC = (0, 7, 6, 1, 2, 5, 4, 3)


def _lut(table, idx):
    r = 0
    for t, v in enumerate(table):
        r = r + v * (idx == t)
    return r


def kernel(x, router_W, route_idx, expert_W, shared_W):
    w_b = expert_W.astype(jnp.bfloat16)

    def body(x_ref, rw_ref, route_ref, w_ref, sw_ref, out_ref,
             xb_ref, gate4_ref, sh_ref,
             acc_r, acc_l, recv_r, recv_l, s_ref, recv2,
             s1r_send, s1r_recv, s1l_send, s1l_recv, s2_send, s2_recv):
        my = lax.axis_index("i")
        z = my // NQ
        q = my % NQ
        cm = _lut(Q2CYC, q)
        cyc_succ = z * NQ + _lut(CYC2Q, (cm + 1) % NQ)
        cyc_pred = z * NQ + _lut(CYC2Q, (cm - 1 + NQ) % NQ)

        xv = x_ref[...]
        scores = jnp.dot(xv, rw_ref[...], preferred_element_type=jnp.float32)
        smax = jnp.max(scores, axis=1, keepdims=True)
        e_ = jnp.exp(scores - smax)
        probs = e_ / jnp.sum(e_, axis=1, keepdims=True)
        route = route_ref[:, 0:1]
        oh128 = route == lax.broadcasted_iota(jnp.int32, (N_TOK, N_EXP), 1)
        p = jnp.sum(probs * oh128.astype(jnp.float32), axis=1, keepdims=True)
        gates = [
            (p * (route == (E_LOCAL * my + j)).astype(jnp.float32))
            for j in range(E_LOCAL)
        ]
        gate = jnp.concatenate(gates, axis=1)
        gate4_ref[...] = gate.astype(jnp.bfloat16).reshape(NZ, NQ * ROWS, E_LOCAL)
        xb_ref[...] = xv.astype(jnp.bfloat16).reshape(NZ, NQ * ROWS, D)

        def sc_partial_half(qq, lo):
            xz = xb_ref[:, pl.ds(qq * ROWS, ROWS), :]
            res = jnp.zeros((SROWS, HH), jnp.float32)
            for j in range(E_LOCAL):
                g = gate4_ref[:, pl.ds(qq * ROWS, ROWS), j:j + 1]
                xg = (xz * g).reshape(SROWS, D)
                res = res + jnp.dot(
                    xg, w_ref[j, :, lo:lo + HH],
                    preferred_element_type=jnp.float32,
                )
            return res

        q0r = _lut(CYC2Q, (cm - 1 + NQ) % NQ)
        q0l = _lut(CYC2Q, (cm + 1) % NQ)
        acc_r[0, :, :, :] = (
            sc_partial_half(q0r, 0).astype(jnp.bfloat16).reshape(NSUB, SUBR, HH)
        )
        acc_l[0, :, :, :] = (
            sc_partial_half(q0l, HH).astype(jnp.bfloat16).reshape(NSUB, SUBR, HH)
        )

        barrier_sem = pltpu.get_barrier_semaphore()
        z_others = tuple(((z + dz) % NZ) * NQ + q for dz in (1, 2, 3))
        for nbr in (cyc_pred, cyc_succ) + z_others:
            pl.semaphore_signal(
                barrier_sem, inc=1,
                device_id=(nbr,), device_id_type=pl.DeviceIdType.MESH,
            )
        pl.semaphore_wait(barrier_sem, 5)

        def mk(ring, s, sub):
            acc, recv, ssend, srecv, dev = (
                (acc_r, recv_r, s1r_send, s1r_recv, cyc_succ)
                if ring == 0
                else (acc_l, recv_l, s1l_send, s1l_recv, cyc_pred)
            )
            return pltpu.make_async_remote_copy(
                src_ref=acc.at[s % 2, sub],
                dst_ref=recv.at[s, sub],
                send_sem=ssend.at[s, sub],
                recv_sem=srecv.at[s, sub],
                device_id=(dev,),
                device_id_type=pl.DeviceIdType.MESH,
            )

        def mk2(sub, dz):
            return pltpu.make_async_remote_copy(
                src_ref=s_ref.at[pl.ds(sub * ROWS, ROWS)],
                dst_ref=recv2.at[dz - 1],
                send_sem=s2_send.at[sub],
                recv_sem=s2_recv.at[dz - 1],
                device_id=(sub * NQ + q,),
                device_id_type=pl.DeviceIdType.MESH,
            )

        for sub in range(NSUB):
            mk(0, 0, sub).start()
            mk(1, 0, sub).start()
        for s in range(NQ - 1):
            slot = s % 2
            qr = _lut(CYC2Q, (cm - s - 2 + 2 * NQ) % NQ)
            ql = _lut(CYC2Q, (cm + s + 2) % NQ)
            part_r = sc_partial_half(qr, 0)
            part_l = sc_partial_half(ql, HH)
            for sub in range(NSUB):
                lo, hi = sub * SUBR, (sub + 1) * SUBR
                rr = mk(0, s, sub)
                ll = mk(1, s, sub)
                rr.wait()
                ll.wait()
                if s < NQ - 2:
                    acc_r[1 - slot, sub, :, :] = (
                        recv_r[s, sub].astype(jnp.float32) + part_r[lo:hi]
                    ).astype(jnp.bfloat16)
                    acc_l[1 - slot, sub, :, :] = (
                        recv_l[s, sub].astype(jnp.float32) + part_l[lo:hi]
                    ).astype(jnp.bfloat16)
                    mk(0, s + 1, sub).start()
                    mk(1, s + 1, sub).start()
                else:
                    s_ref[lo:hi, 0:HH] = (
                        recv_r[s, sub].astype(jnp.float32) + part_r[lo:hi]
                    ).astype(jnp.bfloat16)
                    s_ref[lo:hi, HH:H] = (
                        recv_l[s, sub].astype(jnp.float32) + part_l[lo:hi]
                    ).astype(jnp.bfloat16)
                    dz = (sub - z + NZ) % NZ

                    @pl.when(dz != 0)
                    def _():
                        mk2(sub, dz).start()

        sh_ref[...] = jnp.dot(
            x_ref[pl.ds(my * ROWS, ROWS), :], sw_ref[...],
            preferred_element_type=jnp.float32,
        )
        for sub in range(NZ):
            dz = (sub - z + NZ) % NZ

            @pl.when(dz != 0)
            def _():
                mk2(sub, dz).wait_send()
        for r in range(NZ - 1):
            mk2(0, r + 1).wait_recv()
        out_ref[...] = (
            s_ref[pl.ds(z * ROWS, ROWS), :].astype(jnp.float32)
            + sh_ref[...]
            + recv2[0].astype(jnp.float32)
            + recv2[1].astype(jnp.float32)
            + recv2[2].astype(jnp.float32)
        )

    return pl.pallas_call(
        body,
        out_shape=jax.ShapeDtypeStruct((ROWS, H), jnp.float32),
        in_specs=[
            pl.BlockSpec(memory_space=pltpu.VMEM),
            pl.BlockSpec(memory_space=pltpu.VMEM),
            pl.BlockSpec(memory_space=pltpu.VMEM),
            pl.BlockSpec(memory_space=pltpu.VMEM),
            pl.BlockSpec(memory_space=pltpu.VMEM),
        ],
        out_specs=pl.BlockSpec(memory_space=pltpu.VMEM),
        scratch_shapes=[
            pltpu.VMEM((NZ, NQ * ROWS, D), jnp.bfloat16),
            pltpu.VMEM((NZ, NQ * ROWS, E_LOCAL), jnp.bfloat16),
            pltpu.VMEM((ROWS, H), jnp.float32),
            pltpu.VMEM((2, NSUB, SUBR, HH), jnp.bfloat16),
            pltpu.VMEM((2, NSUB, SUBR, HH), jnp.bfloat16),
            pltpu.VMEM((NQ - 1, NSUB, SUBR, HH), jnp.bfloat16),
            pltpu.VMEM((NQ - 1, NSUB, SUBR, HH), jnp.bfloat16),
            pltpu.VMEM((SROWS, H), jnp.bfloat16),
            pltpu.VMEM((NZ - 1, ROWS, H), jnp.bfloat16),
            pltpu.SemaphoreType.DMA((NQ - 1, NSUB)),
            pltpu.SemaphoreType.DMA((NQ - 1, NSUB)),
            pltpu.SemaphoreType.DMA((NQ - 1, NSUB)),
            pltpu.SemaphoreType.DMA((NQ - 1, NSUB)),
            pltpu.SemaphoreType.DMA((NZ,)),
            pltpu.SemaphoreType.DMA((NZ - 1,)),
        ],
        compiler_params=pltpu.CompilerParams(collective_id=0),
    )(x, router_W, route_idx, w_b, shared_W)
